# baseline (speedup 1.0000x reference)
import functools
import math

import jax
import jax.numpy as jnp
from jax import lax
from jax.experimental import pallas as pl
from jax.experimental.pallas import tpu as pltpu
from jax.experimental.pallas import tpu_sc as plsc

D_MODEL = 64
D_PAD = 128
SCALE = math.sqrt(D_MODEL)
LANES = 16
CHUNK = 128
WCHUNK = 256


@functools.cache
def _build_widen(vocab: int):
    info = plsc.get_sparse_core_info()
    nw = info.num_cores * info.num_subcores
    n_chunks = vocab // WCHUNK
    tail = vocab - n_chunks * WCHUNK
    assert tail % 8 == 0

    mesh = plsc.VectorSubcoreMesh(core_axis_name="c", subcore_axis_name="s")

    @functools.partial(
        pl.kernel,
        out_type=jax.ShapeDtypeStruct((vocab, D_PAD), jnp.float32),
        mesh=mesh,
        scratch_types=[
            pltpu.VMEM((WCHUNK, D_MODEL), jnp.float32),
            pltpu.VMEM((WCHUNK, D_MODEL), jnp.float32),
            pltpu.VMEM((WCHUNK, D_PAD), jnp.float32),
            pltpu.VMEM((WCHUNK, D_PAD), jnp.float32),
            pltpu.SemaphoreType.DMA, pltpu.SemaphoreType.DMA,
            pltpu.SemaphoreType.DMA, pltpu.SemaphoreType.DMA,
        ],
        compiler_params=pltpu.CompilerParams(use_tc_tiling_on_sc=True),
    )
    def widen_kernel(src_hbm, out_hbm, i0, i1, o0, o1,
                     sem_i0, sem_i1, sem_o0, sem_o1):
        wid = lax.axis_index("s") * info.num_cores + lax.axis_index("c")
        ibuf = (i0, i1)
        obuf = (o0, o1)
        isem = (sem_i0, sem_i1)
        osem = (sem_o0, sem_o1)
        n_iter = n_chunks // nw + 1

        def start_stage(c, h):
            pltpu.async_copy(src_hbm.at[pl.ds(c * WCHUNK, WCHUNK)],
                             ibuf[h], isem[h])

        def wait_stage(h):
            pltpu.make_async_copy(src_hbm.at[pl.ds(0, WCHUNK)], ibuf[h],
                                  isem[h]).wait()

        def start_write(c, h):
            pltpu.async_copy(obuf[h], out_hbm.at[pl.ds(c * WCHUNK, WCHUNK)],
                             osem[h])

        def wait_write(h):
            pltpu.make_async_copy(obuf[h], out_hbm.at[pl.ds(0, WCHUNK)],
                                  osem[h]).wait()

        start_stage(wid, 0)

        @pl.when(wid + nw < n_chunks)
        def _():
            start_stage(wid + nw, 1)

        def pair_body(p, carry):
            for h in range(2):
                j = p * 2 + h
                c = wid + j * nw
                valid = c < n_chunks

                @pl.when(valid)
                def _():
                    wait_stage(h)

                    @pl.when(j >= 2)
                    def _():
                        wait_write(h)

                    def copy_body(r, c2):
                        for k in range(D_MODEL // LANES):
                            sl = pl.ds(k * LANES, LANES)
                            obuf[h][r, sl] = ibuf[h][r, sl] * SCALE
                        return c2

                    lax.fori_loop(0, WCHUNK, copy_body, 0)

                    @pl.when(c + 2 * nw < n_chunks)
                    def _():
                        start_stage(c + 2 * nw, h)

                    start_write(c, h)
            return carry

        lax.fori_loop(0, (n_iter + 1) // 2, pair_body, 0)
        wait_write(0)
        wait_write(1)

        if tail:
            @pl.when(wid == nw - 1)
            def _():
                base = n_chunks * WCHUNK
                pltpu.sync_copy(src_hbm.at[pl.ds(base, tail)],
                                i0.at[pl.ds(0, tail)])

                def tail_body(r, c2):
                    for k in range(D_MODEL // LANES):
                        sl = pl.ds(k * LANES, LANES)
                        o0[r, sl] = i0[r, sl] * SCALE
                    return c2

                lax.fori_loop(0, tail, tail_body, 0)
                pltpu.sync_copy(o0.at[pl.ds(0, tail)],
                                out_hbm.at[pl.ds(base, tail)])

    return widen_kernel


@functools.cache
def _build(n_total: int, vocab: int):
    info = plsc.get_sparse_core_info()
    nw = info.num_cores * info.num_subcores
    assert n_total % (nw * CHUNK) == 0
    b_per_w = n_total // nw
    n_chunks = b_per_w // CHUNK
    assert n_chunks % 2 == 0 and n_chunks >= 4

    mesh = plsc.VectorSubcoreMesh(core_axis_name="c", subcore_axis_name="s")

    @functools.partial(
        pl.kernel,
        out_type=jax.ShapeDtypeStruct((n_total, D_MODEL), jnp.float32),
        mesh=mesh,
        scratch_types=[
            pltpu.VMEM((n_chunks, CHUNK), jnp.int32),
            pltpu.VMEM((CHUNK, D_PAD), jnp.float32),
            pltpu.VMEM((CHUNK, D_PAD), jnp.float32),
            pltpu.VMEM((CHUNK, D_MODEL), jnp.float32),
            pltpu.VMEM((CHUNK, D_MODEL), jnp.float32),
            pltpu.SemaphoreType.DMA, pltpu.SemaphoreType.DMA,
            pltpu.SemaphoreType.DMA, pltpu.SemaphoreType.DMA,
        ],
        compiler_params=pltpu.CompilerParams(use_tc_tiling_on_sc=True),
    )
    def emb_kernel(x_hbm, table_hbm, out_hbm, idx_v, g0, g1, s0, s1,
                   sem_g0, sem_g1, sem_s0, sem_s1):
        wid = lax.axis_index("s") * info.num_cores + lax.axis_index("c")
        base = wid * b_per_w
        gbuf = (g0, g1)
        sbuf = (s0, s1)
        gsem = (sem_g0, sem_g1)
        ssem = (sem_s0, sem_s1)

        pltpu.sync_copy(x_hbm.at[wid], idx_v)

        def start_gather(c, b):
            pltpu.async_copy(table_hbm.at[idx_v.at[c]], gbuf[b], gsem[b])

        def start_scatter(c, b):
            pltpu.async_copy(
                sbuf[b], out_hbm.at[pl.ds(base + c * CHUNK, CHUNK)], ssem[b])

        def wait_gather(b):
            pltpu.make_async_copy(table_hbm.at[idx_v.at[0]], gbuf[b],
                                  gsem[b]).wait()

        def wait_scatter(b):
            pltpu.make_async_copy(sbuf[b], out_hbm.at[pl.ds(base, CHUNK)],
                                  ssem[b]).wait()

        start_gather(0, 0)
        start_gather(1, 1)

        def pair_body(i, carry):
            cc = i * 2
            for b in range(2):
                c = cc + b
                wait_gather(b)

                @pl.when(cc > 0)
                def _():
                    wait_scatter(b)

                def compact_body(k, c2):
                    for rr in range(8):
                        r = k * 8 + rr
                        for p in range(D_MODEL // LANES):
                            sl = pl.ds(p * LANES, LANES)
                            sbuf[b][r, sl] = gbuf[b][r, sl]
                    return c2

                lax.fori_loop(0, CHUNK // 8, compact_body, 0)

                @pl.when(c + 2 < n_chunks)
                def _():
                    start_gather(c + 2, b)

                start_scatter(c, b)
            return carry

        lax.fori_loop(0, n_chunks // 2, pair_body, 0)
        wait_scatter(0)
        wait_scatter(1)

    return emb_kernel


def kernel(x, table):
    b, l = x.shape
    xf = x.reshape(-1).astype(jnp.int32)
    n_total = xf.shape[0]
    info = plsc.get_sparse_core_info()
    nw = info.num_cores * info.num_subcores
    x3 = xf.reshape(nw, n_total // (nw * CHUNK), CHUNK)
    table_p = _build_widen(table.shape[0])(table)
    out = _build(n_total, table.shape[0])(x3, table_p)
    return out.reshape(b, l, D_MODEL)


# speedup vs baseline: 1.1655x; 1.1655x over previous
import functools
import math

import jax
import jax.numpy as jnp
from jax import lax
from jax.experimental import pallas as pl
from jax.experimental.pallas import tpu as pltpu
from jax.experimental.pallas import tpu_sc as plsc

D_MODEL = 64
D_PAD = 128
SCALE = math.sqrt(D_MODEL)
LANES = 16
CHUNK = 128


@functools.cache
def _build(n_total: int, vocab: int):
    info = plsc.get_sparse_core_info()
    nw = info.num_cores * info.num_subcores
    assert n_total % (nw * CHUNK) == 0
    b_per_w = n_total // nw
    n_chunks = b_per_w // CHUNK
    assert n_chunks % 4 == 0 and n_chunks >= 8

    mesh = plsc.VectorSubcoreMesh(core_axis_name="c", subcore_axis_name="s")

    @functools.partial(
        pl.kernel,
        out_type=jax.ShapeDtypeStruct((n_total, D_MODEL), jnp.float32),
        mesh=mesh,
        scratch_types=[
            pltpu.VMEM((n_chunks, CHUNK), jnp.int32),
            pltpu.VMEM((CHUNK, D_PAD), jnp.float32),
            pltpu.VMEM((CHUNK, D_PAD), jnp.float32),
            pltpu.VMEM((CHUNK, D_PAD), jnp.float32),
            pltpu.VMEM((CHUNK, D_PAD), jnp.float32),
            pltpu.VMEM((CHUNK, D_MODEL), jnp.float32),
            pltpu.VMEM((CHUNK, D_MODEL), jnp.float32),
            pltpu.SemaphoreType.DMA, pltpu.SemaphoreType.DMA,
            pltpu.SemaphoreType.DMA, pltpu.SemaphoreType.DMA,
            pltpu.SemaphoreType.DMA, pltpu.SemaphoreType.DMA,
        ],
        compiler_params=pltpu.CompilerParams(use_tc_tiling_on_sc=True),
    )
    def emb_kernel(x_hbm, table_hbm, out_hbm, idx_v, g0, g1, g2, g3, s0, s1,
                   sem_g0, sem_g1, sem_g2, sem_g3, sem_s0, sem_s1):
        wid = lax.axis_index("s") * info.num_cores + lax.axis_index("c")
        base = wid * b_per_w
        gbuf = (g0, g1, g2, g3)
        sbuf = (s0, s1)
        gsem = (sem_g0, sem_g1, sem_g2, sem_g3)
        ssem = (sem_s0, sem_s1)

        pltpu.sync_copy(x_hbm.at[wid], idx_v)

        def start_gather(c, b):
            pltpu.async_copy(table_hbm.at[idx_v.at[c]], gbuf[b], gsem[b])

        def start_scatter(c, b):
            pltpu.async_copy(
                sbuf[b], out_hbm.at[pl.ds(base + c * CHUNK, CHUNK)], ssem[b])

        def wait_gather(b):
            pltpu.make_async_copy(table_hbm.at[idx_v.at[0]], gbuf[b],
                                  gsem[b]).wait()

        def wait_scatter(b):
            pltpu.make_async_copy(sbuf[b], out_hbm.at[pl.ds(base, CHUNK)],
                                  ssem[b]).wait()

        for b in range(4):
            start_gather(b, b)

        def quad_body(i, carry):
            cc = i * 4
            for b in range(4):
                c = cc + b
                sb = b % 2
                wait_gather(b)

                @pl.when(c >= 2)
                def _():
                    wait_scatter(sb)

                def scale_body(k, c2):
                    for rr in range(8):
                        r = k * 8 + rr
                        for p in range(D_MODEL // LANES):
                            sl = pl.ds(p * LANES, LANES)
                            sbuf[sb][r, sl] = gbuf[b][r, sl] * SCALE
                    return c2

                lax.fori_loop(0, CHUNK // 8, scale_body, 0)

                @pl.when(c + 4 < n_chunks)
                def _():
                    start_gather(c + 4, b)

                start_scatter(c, sb)
            return carry

        lax.fori_loop(0, n_chunks // 4, quad_body, 0)
        wait_scatter(0)
        wait_scatter(1)

    return emb_kernel


def kernel(x, table):
    b, l = x.shape
    xf = x.reshape(-1).astype(jnp.int32)
    n_total = xf.shape[0]
    info = plsc.get_sparse_core_info()
    nw = info.num_cores * info.num_subcores
    x3 = xf.reshape(nw, n_total // (nw * CHUNK), CHUNK)
    table_p = jnp.pad(table, ((0, 0), (0, D_PAD - D_MODEL)))
    out = _build(n_total, table.shape[0])(x3, table_p)
    return out.reshape(b, l, D_MODEL)
